# Initial kernel scaffold; baseline (speedup 1.0000x reference)
import jax
import jax.numpy as jnp
from jax.experimental import pallas as pl


def kernel(x, embedding):
    raise NotImplementedError("write your pallas kernel here")


# speedup vs baseline: 7.0304x; 7.0304x over previous
import functools

import jax
import jax.numpy as jnp
from jax import lax
from jax.experimental import pallas as pl
from jax.experimental.pallas import tpu as pltpu
from jax.experimental.pallas import tpu_sc as plsc

NC = 2
NS = 16
NW = NC * NS

B = 4096 * 200
D = 128
BPW = B // NW
CH = 128
NCH = BPW // CH

_mesh = plsc.VectorSubcoreMesh(core_axis_name="c", subcore_axis_name="s")


@functools.partial(
    pl.kernel,
    out_type=jax.ShapeDtypeStruct((B, D), jnp.float32),
    mesh=_mesh,
    scratch_types=[
        pltpu.VMEM((NCH, CH), jnp.int32),
        pltpu.VMEM((CH, D), jnp.float32),
        pltpu.SemaphoreType.DMA,
    ],
)
def _gather_kernel(x_hbm, tab_hbm, out_hbm, idx_v, rows_v, sem):
    wid = lax.axis_index("s") * NC + lax.axis_index("c")
    pltpu.sync_copy(x_hbm.at[wid], idx_v)
    base = wid * BPW

    def step(j, carry):
        pltpu.async_copy(tab_hbm.at[idx_v.at[j]], rows_v, sem).wait()
        pltpu.sync_copy(rows_v, out_hbm.at[pl.ds(base + j * CH, CH)])
        return carry

    lax.fori_loop(0, NCH, step, 0)


def kernel(x, embedding):
    xw = x.reshape(NW, NCH, CH).astype(jnp.int32)
    out = _gather_kernel(xw, embedding)
    return out.reshape(x.shape[0], x.shape[1], D)


# speedup vs baseline: 10.7656x; 1.5313x over previous
import functools

import jax
import jax.numpy as jnp
from jax import lax
from jax.experimental import pallas as pl
from jax.experimental.pallas import tpu as pltpu
from jax.experimental.pallas import tpu_sc as plsc

NC = 2
NS = 16
NW = NC * NS

B = 4096 * 200
D = 128
BPW = B // NW
CH = 128
NCH = BPW // CH

_mesh = plsc.VectorSubcoreMesh(core_axis_name="c", subcore_axis_name="s")

V = 10240
VPS = V // NS


@functools.partial(
    pl.kernel,
    out_type=jax.ShapeDtypeStruct((B, D), jnp.float32),
    mesh=_mesh,
    scratch_types=[
        pltpu.VMEM_SHARED((V, D), jnp.float32),
        pltpu.VMEM((NCH, CH), jnp.int32),
        pltpu.VMEM((CH, D), jnp.float32),
        pltpu.SemaphoreType.DMA,
    ],
)
def _gather_kernel(x_hbm, tab_hbm, out_hbm, tab_s, idx_v, rows_v, sem):
    cid = lax.axis_index("c")
    sid = lax.axis_index("s")
    wid = sid * NC + cid

    pltpu.sync_copy(tab_hbm.at[pl.ds(sid * VPS, VPS)],
                    tab_s.at[pl.ds(sid * VPS, VPS)])
    pltpu.sync_copy(x_hbm.at[wid], idx_v)
    plsc.subcore_barrier()

    base = wid * BPW

    def step(j, carry):
        pltpu.async_copy(tab_s.at[idx_v.at[j]], rows_v, sem).wait()
        pltpu.sync_copy(rows_v, out_hbm.at[pl.ds(base + j * CH, CH)])
        return carry

    lax.fori_loop(0, NCH, step, 0)


def kernel(x, embedding):
    xw = x.reshape(NW, NCH, CH).astype(jnp.int32)
    tab = jnp.pad(embedding, ((0, V - embedding.shape[0]), (0, 0)))
    out = _gather_kernel(xw, tab)
    return out.reshape(x.shape[0], x.shape[1], D)


# speedup vs baseline: 16.0223x; 1.4883x over previous
import functools

import jax
import jax.numpy as jnp
from jax import lax
from jax.experimental import pallas as pl
from jax.experimental.pallas import tpu as pltpu
from jax.experimental.pallas import tpu_sc as plsc

NC = 2
NS = 16
NW = NC * NS

B = 4096 * 200
D = 128
BPW = B // NW
CH = 128
NCH = BPW // CH
KB = 40
NBLK = NCH // KB

_mesh = plsc.VectorSubcoreMesh(core_axis_name="c", subcore_axis_name="s")

V = 10240
VPS = V // NS


@functools.partial(
    pl.kernel,
    out_type=jax.ShapeDtypeStruct((B, D), jnp.float32),
    mesh=_mesh,
    scratch_types=[
        pltpu.VMEM_SHARED((V, D), jnp.float32),
        pltpu.VMEM((2, KB, CH), jnp.int32),
        pltpu.VMEM((2, CH, D), jnp.float32),
        pltpu.SemaphoreType.DMA,
        pltpu.SemaphoreType.DMA,
        pltpu.SemaphoreType.DMA,
    ],
)
def _gather_kernel(x_hbm, tab_hbm, out_hbm, tab_s, idx_v, rows_v,
                   isem, gsem, wsem):
    cid = lax.axis_index("c")
    sid = lax.axis_index("s")
    wid = sid * NC + cid

    pltpu.sync_copy(tab_hbm.at[pl.ds(sid * VPS, VPS)],
                    tab_s.at[pl.ds(sid * VPS, VPS)])
    plsc.subcore_barrier()

    base = wid * BPW

    pltpu.async_copy(x_hbm.at[wid, pl.ds(0, KB)], idx_v.at[0], isem)

    def step(j, carry):
        b = j % 2
        blk = j // KB
        ib = blk % 2
        @pl.when(j % KB == 0)
        def _():
            pltpu.make_async_copy(x_hbm.at[wid, pl.ds(0, KB)],
                                  idx_v.at[ib], isem).wait()
            @pl.when(blk + 1 < NBLK)
            def _():
                pltpu.async_copy(x_hbm.at[wid, pl.ds((blk + 1) * KB, KB)],
                                 idx_v.at[(blk + 1) % 2], isem)
        @pl.when(j >= 2)
        def _():
            pltpu.make_async_copy(
                rows_v.at[b], out_hbm.at[pl.ds(base, CH)], wsem).wait()
        pltpu.async_copy(tab_s.at[idx_v.at[ib, j % KB]], rows_v.at[b],
                         gsem).wait()
        pltpu.async_copy(rows_v.at[b], out_hbm.at[pl.ds(base + j * CH, CH)],
                         wsem)
        return carry

    lax.fori_loop(0, NCH, step, 0)

    pltpu.make_async_copy(rows_v.at[0], out_hbm.at[pl.ds(base, CH)],
                          wsem).wait()
    pltpu.make_async_copy(rows_v.at[1], out_hbm.at[pl.ds(base, CH)],
                          wsem).wait()


def kernel(x, embedding):
    xw = x.reshape(NW, NCH, CH).astype(jnp.int32)
    tab = jnp.pad(embedding, ((0, V - embedding.shape[0]), (0, 0)))
    out = _gather_kernel(xw, tab)
    return out.reshape(x.shape[0], x.shape[1], D)
